# baseline (speedup 1.0000x reference)
import jax
import jax.numpy as jnp
from jax import lax
from jax.experimental import pallas as pl
from jax.experimental.pallas import tpu as pltpu

D_MODEL = 1024
BATCH = 16
HALF = BATCH // 2


def _body(idx_smem, hs_hbm, out_hbm, buf, sem_a, sem_b, sem_o):
    for b in range(BATCH):
        pltpu.make_async_copy(
            hs_hbm.at[pl.ds(idx_smem[b], 1)],
            buf.at[pl.ds(b, 1)],
            sem_a if b < HALF else sem_b,
        ).start()

    def _normalize_half(h, sem):
        sl = pl.ds(h * HALF, HALF)
        pltpu.make_async_copy(hs_hbm.at[pl.ds(0, HALF)], buf.at[sl], sem).wait()
        x = buf[sl]
        ss = jnp.sum(x * x, axis=1, keepdims=True)
        buf[sl] = x * lax.rsqrt(jnp.maximum(ss, 1e-24))
        pltpu.make_async_copy(buf.at[sl], out_hbm.at[sl], sem_o).start()

    _normalize_half(0, sem_a)
    _normalize_half(1, sem_b)
    pltpu.make_async_copy(buf, out_hbm, sem_o).wait()


@jax.jit
def _pooler(hs, idx):
    return pl.pallas_call(
        _body,
        in_specs=[
            pl.BlockSpec(memory_space=pltpu.MemorySpace.SMEM),
            pl.BlockSpec(memory_space=pltpu.MemorySpace.HBM),
        ],
        out_specs=pl.BlockSpec(memory_space=pltpu.MemorySpace.HBM),
        scratch_shapes=[
            pltpu.VMEM((BATCH, D_MODEL), jnp.float32),
            pltpu.SemaphoreType.DMA,
            pltpu.SemaphoreType.DMA,
            pltpu.SemaphoreType.DMA,
        ],
        out_shape=jax.ShapeDtypeStruct((BATCH, D_MODEL), jnp.float32),
    )(idx, hs)


def kernel(hidden_states, last_token_indices):
    hs = hidden_states.astype(jnp.float32)
    idx = last_token_indices.astype(jnp.int32)
    return _pooler(hs, idx)


# speedup vs baseline: 1.0329x; 1.0329x over previous
import jax
import jax.numpy as jnp
from jax import lax
from jax.experimental import pallas as pl
from jax.experimental.pallas import tpu as pltpu

D_MODEL = 1024
BATCH = 16


def _body(idx_smem, hs_hbm, out_vmem, buf, sem):
    for b in range(BATCH):
        pltpu.make_async_copy(
            hs_hbm.at[pl.ds(idx_smem[b], 1)], buf.at[pl.ds(b, 1)], sem
        ).start()
    pltpu.make_async_copy(hs_hbm.at[pl.ds(0, BATCH)], buf, sem).wait()
    x = buf[...]
    ss = jnp.sum(x * x, axis=1, keepdims=True)
    out_vmem[...] = x * lax.rsqrt(jnp.maximum(ss, 1e-24))


@jax.jit
def _pooler(hs, idx):
    return pl.pallas_call(
        _body,
        in_specs=[
            pl.BlockSpec(memory_space=pltpu.MemorySpace.SMEM),
            pl.BlockSpec(memory_space=pltpu.MemorySpace.HBM),
        ],
        out_specs=pl.BlockSpec(memory_space=pltpu.MemorySpace.VMEM),
        scratch_shapes=[
            pltpu.VMEM((BATCH, D_MODEL), jnp.float32),
            pltpu.SemaphoreType.DMA,
        ],
        out_shape=jax.ShapeDtypeStruct((BATCH, D_MODEL), jnp.float32),
    )(idx, hs)


def kernel(hidden_states, last_token_indices):
    hs = hidden_states.astype(jnp.float32)
    idx = last_token_indices.astype(jnp.int32)
    return _pooler(hs, idx)
